# Initial kernel scaffold; baseline (speedup 1.0000x reference)
import jax
import jax.numpy as jnp
from jax.experimental import pallas as pl


def kernel(input, edge_index, w, a_src_dst):
    raise NotImplementedError("write your pallas kernel here")


# speedup vs baseline: 3.2520x; 3.2520x over previous
import functools

import jax
import jax.numpy as jnp
from jax import lax
from jax.experimental import pallas as pl
from jax.experimental.pallas import tpu as pltpu
from jax.experimental.pallas import tpu_sc as plsc

N = 10000
E = 160000
F_IN = 256
F_OUT = 256
N_HEAD = 4
C = 128
NCHUNK = (N_HEAD * F_OUT) // C
B = 80
L = 16

NODE_SLICE = 624
LAST_SLICE = N - 15 * NODE_SLICE


def _matmul_chunks_kernel(x_ref, w_ref, o_ref):
    o_ref[0] = jnp.dot(x_ref[...], w_ref[...],
                       preferred_element_type=jnp.float32)


def _alphas_kernel(b_ref, x_ref, o_ref):
    o_ref[...] = lax.dot_general(b_ref[...], x_ref[...],
                                 (((1,), (1,)), ((), ())),
                                 preferred_element_type=jnp.float32)


def _bcast16(v, j):
    idx = jnp.full((16,), j, dtype=jnp.int32)
    dnums = lax.GatherDimensionNumbers(
        offset_dims=(), collapsed_slice_dims=(0,), start_index_map=(0,))
    return lax.gather(v, idx[:, None], dnums, (1,),
                      mode=lax.GatherScatterMode.PROMISE_IN_BOUNDS)


def _sc_gat(hc, alphas, src, dst):
    mesh = plsc.VectorSubcoreMesh(core_axis_name="c", subcore_axis_name="s")
    n_batch = (E // 16) // B

    @functools.partial(
        pl.kernel,
        out_type=jax.ShapeDtypeStruct((N_HEAD, N, F_OUT), jnp.float32),
        mesh=mesh,
        compiler_params=pltpu.CompilerParams(needs_layout_passes=False),
        scratch_types=[
            pltpu.VMEM((N,), jnp.float32),
            pltpu.VMEM((N,), jnp.float32),
            pltpu.VMEM((1, B), jnp.int32),
            pltpu.VMEM((1, B), jnp.int32),
            pltpu.VMEM((B, C), jnp.float32),
            pltpu.VMEM((B,), jnp.float32),
            pltpu.VMEM((16, C), jnp.float32),
            pltpu.VMEM((16, C), jnp.float32),
            pltpu.VMEM((LAST_SLICE,), jnp.float32),
            pltpu.VMEM((LAST_SLICE,), jnp.float32),
            pltpu.VMEM_SHARED((N, C), jnp.float32),
            pltpu.VMEM_SHARED((N,), jnp.float32),
            pltpu.SemaphoreType.DMA,
        ],
    )
    def kern(hc_ref, al_ref, src_ref, dst_ref, out_ref,
             as_v, ad_v, sidx, didx, rows, wbuf, zblk, fblk, zcol, rsv,
             acc, rsum, sem):
        tid = lax.axis_index("s")
        core = lax.axis_index("c")
        ebase = tid * (E // 16)
        lo = tid * NODE_SLICE
        cnt = jnp.where(tid == 15, LAST_SLICE, NODE_SLICE)
        nz16 = cnt // 16

        zeros = jnp.zeros((16,), jnp.float32)
        for r in range(16):
            for q in range(C // L):
                zblk[r, pl.ds(q * L, L)] = zeros
        def zc_body(i, _):
            zcol[pl.ds(i * L, L)] = zeros
            return 0
        lax.fori_loop(0, LAST_SLICE // L, zc_body, 0)

        for p in range(4):
            c = core * 4 + p
            h = c // 2
            f = p % 2

            pltpu.sync_copy(al_ref.at[2 * h], as_v)
            pltpu.sync_copy(al_ref.at[2 * h + 1], ad_v)

            def zero_body(i, _):
                pltpu.sync_copy(zblk, acc.at[pl.ds(lo + i * 16, 16)])
                return 0
            lax.fori_loop(0, nz16, zero_body, 0)
            if f == 0:
                pltpu.sync_copy(zcol.at[pl.ds(0, NODE_SLICE)],
                                rsum.at[pl.ds(lo, NODE_SLICE)])

                @pl.when(tid == 15)
                def _():
                    pltpu.sync_copy(
                        zcol.at[pl.ds(0, LAST_SLICE - NODE_SLICE)],
                        rsum.at[pl.ds(16 * NODE_SLICE,
                                      LAST_SLICE - NODE_SLICE)])
            plsc.subcore_barrier()

            def batch_body(i, _):
                eb = ebase + i * B
                pltpu.sync_copy(src_ref.at[pl.ds(eb, B)], sidx.at[0])
                pltpu.sync_copy(dst_ref.at[pl.ds(eb, B)], didx.at[0])
                gat = pltpu.async_copy(hc_ref.at[c].at[didx.at[0]], rows, sem)
                for k in range(B // L):
                    s16 = sidx[0, pl.ds(k * L, L)]
                    d16 = didx[0, pl.ds(k * L, L)]
                    av = plsc.load_gather(as_v, [s16])
                    dv = plsc.load_gather(ad_v, [d16])
                    lg = av + dv
                    lr = jnp.where(lg >= 0, lg, 0.2 * lg)
                    wbuf[pl.ds(k * L, L)] = jnp.exp(-lr)
                gat.wait()
                def scale_body(k, _):
                    w16 = wbuf[pl.ds(k * L, L)]
                    for j in range(L):
                        wj = _bcast16(w16, j)
                        for q in range(C // L):
                            rows[k * L + j, pl.ds(q * L, L)] = (
                                rows[k * L + j, pl.ds(q * L, L)] * wj)
                    return 0
                lax.fori_loop(0, B // L, scale_body, 0)
                pltpu.sync_copy(rows, acc.at[sidx.at[0]], add=True)
                if f == 0:
                    pltpu.sync_copy(wbuf, rsum.at[sidx.at[0]], add=True)
                return 0
            lax.fori_loop(0, n_batch, batch_body, 0)
            plsc.subcore_barrier()

            pltpu.sync_copy(rsum.at[pl.ds(lo, NODE_SLICE)],
                            rsv.at[pl.ds(0, NODE_SLICE)])
            @pl.when(tid == 15)
            def _():
                pltpu.sync_copy(
                    rsum.at[pl.ds(15 * NODE_SLICE + NODE_SLICE,
                                  LAST_SLICE - NODE_SLICE)],
                    rsv.at[pl.ds(NODE_SLICE, LAST_SLICE - NODE_SLICE)])

            def flush_body(i, _):
                pltpu.sync_copy(acc.at[pl.ds(lo + i * 16, 16)], fblk)
                r16 = 1.0 / rsv[pl.ds(i * L, L)]
                for j in range(L):
                    rj = _bcast16(r16, j)
                    for q in range(C // L):
                        fblk[j, pl.ds(q * L, L)] = (
                            fblk[j, pl.ds(q * L, L)] * rj)
                pltpu.sync_copy(
                    fblk,
                    out_ref.at[h].at[pl.ds(lo + i * 16, 16),
                                     pl.ds(f * C, C)])
                return 0
            lax.fori_loop(0, nz16, flush_body, 0)
            plsc.subcore_barrier()

    return kern(hc, alphas, src, dst)


def kernel(input, edge_index, w, a_src_dst):
    x = input
    wc = jnp.transpose(w, (1, 0, 2)).reshape(F_IN, N_HEAD * F_OUT)
    a_src = a_src_dst[:, :F_OUT, 0]
    a_dst = a_src_dst[:, F_OUT:, 0]
    bs = jnp.einsum("hij,hj->hi", w, a_src)
    bd = jnp.einsum("hij,hj->hi", w, a_dst)
    bsd = jnp.stack([bs, bd], axis=1).reshape(2 * N_HEAD, F_IN)

    hc = pl.pallas_call(
        _matmul_chunks_kernel,
        grid=(NCHUNK, N // 1000),
        in_specs=[
            pl.BlockSpec((1000, F_IN), lambda c, n: (n, 0)),
            pl.BlockSpec((F_IN, C), lambda c, n: (0, c)),
        ],
        out_specs=pl.BlockSpec((1, 1000, C), lambda c, n: (c, n, 0)),
        out_shape=jax.ShapeDtypeStruct((NCHUNK, N, C), jnp.float32),
    )(x, wc)

    alphas = pl.pallas_call(
        _alphas_kernel,
        in_specs=[
            pl.BlockSpec((2 * N_HEAD, F_IN), lambda: (0, 0)),
            pl.BlockSpec((N, F_IN), lambda: (0, 0)),
        ],
        out_specs=pl.BlockSpec((2 * N_HEAD, N), lambda: (0, 0)),
        out_shape=jax.ShapeDtypeStruct((2 * N_HEAD, N), jnp.float32),
    )(bsd, x)

    src = edge_index[0]
    dst = edge_index[1]
    return _sc_gat(hc, alphas, src, dst)


# speedup vs baseline: 3.5726x; 1.0986x over previous
import functools

import jax
import jax.numpy as jnp
from jax import lax
from jax.experimental import pallas as pl
from jax.experimental.pallas import tpu as pltpu
from jax.experimental.pallas import tpu_sc as plsc

N = 10000
E = 160000
F_IN = 256
F_OUT = 256
N_HEAD = 4
C = 128
NCHUNK = (N_HEAD * F_OUT) // C
B = 80
L = 16
NB = (E // 16) // B

NODE_SLICE = 624
LAST_SLICE = N - 15 * NODE_SLICE
ZR = 24


def _matmul_chunks_kernel(x_ref, w_ref, o_ref):
    o_ref[0] = jnp.dot(x_ref[...], w_ref[...],
                       preferred_element_type=jnp.float32)


def _alphas_kernel(b_ref, x_ref, o_ref):
    o_ref[...] = lax.dot_general(b_ref[...], x_ref[...],
                                 (((1,), (1,)), ((), ())),
                                 preferred_element_type=jnp.float32)


def _bcast16(v, j):
    idx = jnp.full((16,), j, dtype=jnp.int32)
    dnums = lax.GatherDimensionNumbers(
        offset_dims=(), collapsed_slice_dims=(0,), start_index_map=(0,))
    return lax.gather(v, idx[:, None], dnums, (1,),
                      mode=lax.GatherScatterMode.PROMISE_IN_BOUNDS)


def _sc_gat(hc, alphas, src, dst):
    mesh = plsc.VectorSubcoreMesh(core_axis_name="c", subcore_axis_name="s")

    @functools.partial(
        pl.kernel,
        out_type=jax.ShapeDtypeStruct((N_HEAD, N, F_OUT), jnp.float32),
        mesh=mesh,
        compiler_params=pltpu.CompilerParams(needs_layout_passes=False),
        scratch_types=[
            pltpu.VMEM((N,), jnp.float32),
            pltpu.VMEM((N,), jnp.float32),
            pltpu.VMEM((1, B), jnp.int32),
            pltpu.VMEM((1, B), jnp.int32),
            pltpu.VMEM((1, B), jnp.int32),
            pltpu.VMEM((1, B), jnp.int32),
            pltpu.VMEM((B,), jnp.float32),
            pltpu.VMEM((B,), jnp.float32),
            pltpu.VMEM((B, C), jnp.float32),
            pltpu.VMEM((B, C), jnp.float32),
            pltpu.VMEM((16, C), jnp.float32),
            pltpu.VMEM((16, C), jnp.float32),
            pltpu.VMEM((LAST_SLICE,), jnp.float32),
            pltpu.VMEM((LAST_SLICE,), jnp.float32),
            pltpu.VMEM_SHARED((N, C), jnp.float32),
            pltpu.VMEM_SHARED((N,), jnp.float32),
            pltpu.SemaphoreType.DMA,
            pltpu.SemaphoreType.DMA,
        ],
    )
    def kern(hc_ref, al_ref, src_ref, dst_ref, out_ref,
             as_v, ad_v, sidx0, sidx1, didx0, didx1, wb0, wb1,
             rows0, rows1, zblk, fblk, zcol, rsv, acc, rsum, sem0, sem1):
        tid = lax.axis_index("s")
        core = lax.axis_index("c")
        ebase = tid * (E // 16)
        lo = tid * NODE_SLICE
        cnt = jnp.where(tid == 15, LAST_SLICE, NODE_SLICE)
        nz16 = cnt // 16

        zeros = jnp.zeros((16,), jnp.float32)
        for r in range(16):
            for q in range(C // L):
                zblk[r, pl.ds(q * L, L)] = zeros
        def zc_body(i, _):
            zcol[pl.ds(i * L, L)] = zeros
            return 0
        lax.fori_loop(0, LAST_SLICE // L, zc_body, 0)

        def load_idx(i, si, di):
            pltpu.sync_copy(src_ref.at[pl.ds(ebase + i * B, B)], si.at[0])
            pltpu.sync_copy(dst_ref.at[pl.ds(ebase + i * B, B)], di.at[0])

        def process(c, f, si, di, wb, rows):
            def group(k, _):
                s16 = si[0, pl.ds(k * L, L)]
                d16 = di[0, pl.ds(k * L, L)]
                av = plsc.load_gather(as_v, [s16])
                dv = plsc.load_gather(ad_v, [d16])
                lg = av + dv
                lr = jnp.where(lg >= 0, lg, 0.2 * lg)
                w16 = jnp.exp(-lr)
                if f == 0:
                    wb[pl.ds(k * L, L)] = w16
                for j16 in range(L):
                    wj = _bcast16(w16, j16)
                    e = k * L + j16
                    for q in range(C // L):
                        rows[e, pl.ds(q * L, L)] = (
                            rows[e, pl.ds(q * L, L)] * wj)
                return 0
            lax.fori_loop(0, B // L, group, 0)
            pltpu.sync_copy(rows, acc.at[si.at[0]], add=True)
            if f == 0:
                pltpu.sync_copy(wb, rsum.at[si.at[0]], add=True)

        for p in range(4):
            c = core * 4 + p
            h = c // 2
            f = p % 2

            if f == 0:
                pltpu.sync_copy(al_ref.at[2 * h], as_v)
                pltpu.sync_copy(al_ref.at[2 * h + 1], ad_v)

            def zero_body(i, _):
                pltpu.sync_copy(zblk, acc.at[pl.ds(lo + i * 16, 16)])
                return 0
            lax.fori_loop(0, nz16, zero_body, 0)
            if f == 0:
                pltpu.sync_copy(zcol.at[pl.ds(0, NODE_SLICE)],
                                rsum.at[pl.ds(lo, NODE_SLICE)])

                @pl.when(tid == 15)
                def _():
                    pltpu.sync_copy(
                        zcol.at[pl.ds(0, LAST_SLICE - NODE_SLICE)],
                        rsum.at[pl.ds(16 * NODE_SLICE,
                                      LAST_SLICE - NODE_SLICE)])
            plsc.subcore_barrier()

            load_idx(0, sidx0, didx0)

            def pair_body(i2, _):
                j = i2 * 2
                load_idx(j + 1, sidx1, didx1)
                g0 = pltpu.async_copy(hc_ref.at[c].at[didx0.at[0]],
                                      rows0, sem0)
                g1 = pltpu.async_copy(hc_ref.at[c].at[didx1.at[0]],
                                      rows1, sem1)
                g0.wait()
                process(c, f, sidx0, didx0, wb0, rows0)
                g1.wait()
                process(c, f, sidx1, didx1, wb1, rows1)
                load_idx(j + 2, sidx0, didx0)
                return 0
            lax.fori_loop(0, (NB - 1) // 2, pair_body, 0)
            ge = pltpu.async_copy(hc_ref.at[c].at[didx0.at[0]], rows0, sem0)
            ge.wait()
            process(c, f, sidx0, didx0, wb0, rows0)
            plsc.subcore_barrier()

            pltpu.sync_copy(rsum.at[pl.ds(lo, NODE_SLICE)],
                            rsv.at[pl.ds(0, NODE_SLICE)])
            @pl.when(tid == 15)
            def _():
                pltpu.sync_copy(
                    rsum.at[pl.ds(16 * NODE_SLICE,
                                  LAST_SLICE - NODE_SLICE)],
                    rsv.at[pl.ds(NODE_SLICE, LAST_SLICE - NODE_SLICE)])

            def flush_body(i, _):
                pltpu.sync_copy(acc.at[pl.ds(lo + i * 16, 16)], fblk)
                r16 = 1.0 / rsv[pl.ds(i * L, L)]
                for j16 in range(L):
                    rj = _bcast16(r16, j16)
                    for q in range(C // L):
                        fblk[j16, pl.ds(q * L, L)] = (
                            fblk[j16, pl.ds(q * L, L)] * rj)
                pltpu.sync_copy(
                    fblk,
                    out_ref.at[h].at[pl.ds(lo + i * 16, 16),
                                     pl.ds(f * C, C)])
                return 0
            lax.fori_loop(0, nz16, flush_body, 0)
            plsc.subcore_barrier()

    return kern(hc, alphas, src, dst)


def kernel(input, edge_index, w, a_src_dst):
    x = input
    wc = jnp.transpose(w, (1, 0, 2)).reshape(F_IN, N_HEAD * F_OUT)
    a_src = a_src_dst[:, :F_OUT, 0]
    a_dst = a_src_dst[:, F_OUT:, 0]
    bs = jnp.einsum("hij,hj->hi", w, a_src)
    bd = jnp.einsum("hij,hj->hi", w, a_dst)
    bsd = jnp.stack([bs, bd], axis=1).reshape(2 * N_HEAD, F_IN)

    hc = pl.pallas_call(
        _matmul_chunks_kernel,
        grid=(NCHUNK, N // 1000),
        in_specs=[
            pl.BlockSpec((1000, F_IN), lambda c, n: (n, 0)),
            pl.BlockSpec((F_IN, C), lambda c, n: (0, c)),
        ],
        out_specs=pl.BlockSpec((1, 1000, C), lambda c, n: (c, n, 0)),
        out_shape=jax.ShapeDtypeStruct((NCHUNK, N, C), jnp.float32),
    )(x, wc)

    alphas = pl.pallas_call(
        _alphas_kernel,
        in_specs=[
            pl.BlockSpec((2 * N_HEAD, F_IN), lambda: (0, 0)),
            pl.BlockSpec((N, F_IN), lambda: (0, 0)),
        ],
        out_specs=pl.BlockSpec((2 * N_HEAD, N), lambda: (0, 0)),
        out_shape=jax.ShapeDtypeStruct((2 * N_HEAD, N), jnp.float32),
    )(bsd, x)

    return _sc_gat(hc, alphas, edge_index[0], edge_index[1])


# speedup vs baseline: 4.0948x; 1.1462x over previous
import functools

import jax
import jax.numpy as jnp
from jax import lax
from jax.experimental import pallas as pl
from jax.experimental.pallas import tpu as pltpu
from jax.experimental.pallas import tpu_sc as plsc

N = 10000
E = 160000
F_IN = 256
F_OUT = 256
N_HEAD = 4
C = 128
NCHUNK = (N_HEAD * F_OUT) // C
B = 80
L = 16
NB = (E // 16) // B

NODE_SLICE = 624
LAST_SLICE = N - 15 * NODE_SLICE
ZR = 24


def _matmul_chunks_kernel(x_ref, w_ref, o_ref):
    o_ref[0] = jnp.dot(x_ref[...], w_ref[...],
                       preferred_element_type=jnp.float32)


def _alphas_kernel(b_ref, x_ref, o_ref):
    o_ref[...] = lax.dot_general(b_ref[...], x_ref[...],
                                 (((1,), (1,)), ((), ())),
                                 preferred_element_type=jnp.float32)


def _bcast16(v, j):
    idx = jnp.full((16,), j, dtype=jnp.int32)
    dnums = lax.GatherDimensionNumbers(
        offset_dims=(), collapsed_slice_dims=(0,), start_index_map=(0,))
    return lax.gather(v, idx[:, None], dnums, (1,),
                      mode=lax.GatherScatterMode.PROMISE_IN_BOUNDS)


def _sc_gat(hc, alphas, src, dst):
    mesh = plsc.VectorSubcoreMesh(core_axis_name="c", subcore_axis_name="s")

    @functools.partial(
        pl.kernel,
        out_type=jax.ShapeDtypeStruct((N_HEAD, N, F_OUT), jnp.float32),
        mesh=mesh,
        compiler_params=pltpu.CompilerParams(needs_layout_passes=False),
        scratch_types=[
            pltpu.VMEM((N,), jnp.float32),
            pltpu.VMEM((N,), jnp.float32),
            pltpu.VMEM((1, B), jnp.int32),
            pltpu.VMEM((1, B), jnp.int32),
            pltpu.VMEM((1, B), jnp.int32),
            pltpu.VMEM((1, B), jnp.int32),
            pltpu.VMEM((B,), jnp.float32),
            pltpu.VMEM((B,), jnp.float32),
            pltpu.VMEM((B, C), jnp.float32),
            pltpu.VMEM((B, C), jnp.float32),
            pltpu.VMEM((16, C), jnp.float32),
            pltpu.VMEM((16, C), jnp.float32),
            pltpu.VMEM((LAST_SLICE,), jnp.float32),
            pltpu.VMEM((LAST_SLICE,), jnp.float32),
            pltpu.VMEM_SHARED((N, C), jnp.float32),
            pltpu.VMEM_SHARED((N,), jnp.float32),
            pltpu.SemaphoreType.DMA,
            pltpu.SemaphoreType.DMA,
            pltpu.SemaphoreType.DMA,
            pltpu.SemaphoreType.DMA,
        ],
    )
    def kern(hc_ref, al_ref, src_ref, dst_ref, out_ref,
             as_v, ad_v, sidx0, sidx1, didx0, didx1, wb0, wb1,
             rows0, rows1, zblk, fblk, zcol, rsv, acc, rsum,
             sem0, sem1, sems0, sems1):
        tid = lax.axis_index("s")
        core = lax.axis_index("c")
        ebase = tid * (E // 16)
        lo = tid * NODE_SLICE
        cnt = jnp.where(tid == 15, LAST_SLICE, NODE_SLICE)
        nz16 = cnt // 16

        zeros = jnp.zeros((16,), jnp.float32)
        for r in range(16):
            for q in range(C // L):
                zblk[r, pl.ds(q * L, L)] = zeros
        def zc_body(i, _):
            zcol[pl.ds(i * L, L)] = zeros
            return 0
        lax.fori_loop(0, LAST_SLICE // L, zc_body, 0)

        def load_idx(i, si, di):
            pltpu.sync_copy(src_ref.at[pl.ds(ebase + i * B, B)], si.at[0])
            pltpu.sync_copy(dst_ref.at[pl.ds(ebase + i * B, B)], di.at[0])

        def process(c, f, si, di, wb, rows, sems):
            def group(k, _):
                s16 = si[0, pl.ds(k * L, L)]
                d16 = di[0, pl.ds(k * L, L)]
                av = plsc.load_gather(as_v, [s16])
                dv = plsc.load_gather(ad_v, [d16])
                lg = av + dv
                lr = jnp.where(lg >= 0, lg, 0.2 * lg)
                w16 = jnp.exp(-lr)
                if f == 0:
                    wb[pl.ds(k * L, L)] = w16
                for j16 in range(L):
                    wj = _bcast16(w16, j16)
                    e = k * L + j16
                    for q in range(C // L):
                        rows[e, pl.ds(q * L, L)] = (
                            rows[e, pl.ds(q * L, L)] * wj)
                return 0
            lax.fori_loop(0, B // L, group, 0)
            d = pltpu.async_copy(rows, acc.at[si.at[0]], sems, add=True)
            if f == 0:
                pltpu.sync_copy(wb, rsum.at[si.at[0]], add=True)
            return d

        for p in range(4):
            c = core * 4 + p
            h = c // 2
            f = p % 2

            if f == 0:
                pltpu.sync_copy(al_ref.at[2 * h], as_v)
                pltpu.sync_copy(al_ref.at[2 * h + 1], ad_v)

            def zero_body(i, _):
                pltpu.sync_copy(zblk, acc.at[pl.ds(lo + i * 16, 16)])
                return 0
            lax.fori_loop(0, nz16, zero_body, 0)
            if f == 0:
                pltpu.sync_copy(zcol.at[pl.ds(0, NODE_SLICE)],
                                rsum.at[pl.ds(lo, NODE_SLICE)])

                @pl.when(tid == 15)
                def _():
                    pltpu.sync_copy(
                        zcol.at[pl.ds(0, LAST_SLICE - NODE_SLICE)],
                        rsum.at[pl.ds(16 * NODE_SLICE,
                                      LAST_SLICE - NODE_SLICE)])
            plsc.subcore_barrier()

            load_idx(0, sidx0, didx0)

            def pair_body(i2, _):
                j = i2 * 2
                load_idx(j + 1, sidx1, didx1)
                g0 = pltpu.async_copy(hc_ref.at[c].at[didx0.at[0]],
                                      rows0, sem0)
                g1 = pltpu.async_copy(hc_ref.at[c].at[didx1.at[0]],
                                      rows1, sem1)
                g0.wait()
                d0 = process(c, f, sidx0, didx0, wb0, rows0, sems0)
                g1.wait()
                d1 = process(c, f, sidx1, didx1, wb1, rows1, sems1)
                d0.wait()
                load_idx(j + 2, sidx0, didx0)
                d1.wait()
                return 0
            lax.fori_loop(0, (NB - 1) // 2, pair_body, 0)
            ge = pltpu.async_copy(hc_ref.at[c].at[didx0.at[0]], rows0, sem0)
            ge.wait()
            process(c, f, sidx0, didx0, wb0, rows0, sems0).wait()
            plsc.subcore_barrier()

            pltpu.sync_copy(rsum.at[pl.ds(lo, NODE_SLICE)],
                            rsv.at[pl.ds(0, NODE_SLICE)])
            @pl.when(tid == 15)
            def _():
                pltpu.sync_copy(
                    rsum.at[pl.ds(16 * NODE_SLICE,
                                  LAST_SLICE - NODE_SLICE)],
                    rsv.at[pl.ds(NODE_SLICE, LAST_SLICE - NODE_SLICE)])

            def flush_body(i, _):
                pltpu.sync_copy(acc.at[pl.ds(lo + i * 16, 16)], fblk)
                r16 = 1.0 / rsv[pl.ds(i * L, L)]
                for j16 in range(L):
                    rj = _bcast16(r16, j16)
                    for q in range(C // L):
                        fblk[j16, pl.ds(q * L, L)] = (
                            fblk[j16, pl.ds(q * L, L)] * rj)
                pltpu.sync_copy(
                    fblk,
                    out_ref.at[h].at[pl.ds(lo + i * 16, 16),
                                     pl.ds(f * C, C)])
                return 0
            lax.fori_loop(0, nz16, flush_body, 0)
            plsc.subcore_barrier()

    return kern(hc, alphas, src, dst)


def kernel(input, edge_index, w, a_src_dst):
    x = input
    wc = jnp.transpose(w, (1, 0, 2)).reshape(F_IN, N_HEAD * F_OUT)
    a_src = a_src_dst[:, :F_OUT, 0]
    a_dst = a_src_dst[:, F_OUT:, 0]
    bs = jnp.einsum("hij,hj->hi", w, a_src)
    bd = jnp.einsum("hij,hj->hi", w, a_dst)
    bsd = jnp.stack([bs, bd], axis=1).reshape(2 * N_HEAD, F_IN)

    hc = pl.pallas_call(
        _matmul_chunks_kernel,
        grid=(NCHUNK, N // 1000),
        in_specs=[
            pl.BlockSpec((1000, F_IN), lambda c, n: (n, 0)),
            pl.BlockSpec((F_IN, C), lambda c, n: (0, c)),
        ],
        out_specs=pl.BlockSpec((1, 1000, C), lambda c, n: (c, n, 0)),
        out_shape=jax.ShapeDtypeStruct((NCHUNK, N, C), jnp.float32),
    )(x, wc)

    alphas = pl.pallas_call(
        _alphas_kernel,
        in_specs=[
            pl.BlockSpec((2 * N_HEAD, F_IN), lambda: (0, 0)),
            pl.BlockSpec((N, F_IN), lambda: (0, 0)),
        ],
        out_specs=pl.BlockSpec((2 * N_HEAD, N), lambda: (0, 0)),
        out_shape=jax.ShapeDtypeStruct((2 * N_HEAD, N), jnp.float32),
    )(bsd, x)

    return _sc_gat(hc, alphas, edge_index[0], edge_index[1])


# speedup vs baseline: 4.1352x; 1.0099x over previous
import functools

import jax
import jax.numpy as jnp
from jax import lax
from jax.experimental import pallas as pl
from jax.experimental.pallas import tpu as pltpu
from jax.experimental.pallas import tpu_sc as plsc

N = 10000
E = 160000
F_IN = 256
F_OUT = 256
N_HEAD = 4
C = 128
NCHUNK = (N_HEAD * F_OUT) // C
B = 80
L = 16
NB = (E // 16) // B

NODE_SLICE = 624
LAST_SLICE = N - 15 * NODE_SLICE
ZR = 24


def _matmul_chunks_kernel(x_ref, w_ref, o_ref):
    o_ref[0] = jnp.dot(x_ref[...], w_ref[...],
                       preferred_element_type=jnp.float32)


def _alphas_kernel(b_ref, x_ref, o_ref):
    o_ref[...] = lax.dot_general(b_ref[...], x_ref[...],
                                 (((1,), (1,)), ((), ())),
                                 preferred_element_type=jnp.float32)


def _bcast16(v, j):
    idx = jnp.full((16,), j, dtype=jnp.int32)
    dnums = lax.GatherDimensionNumbers(
        offset_dims=(), collapsed_slice_dims=(0,), start_index_map=(0,))
    return lax.gather(v, idx[:, None], dnums, (1,),
                      mode=lax.GatherScatterMode.PROMISE_IN_BOUNDS)


def _sc_gat(hc, alphas, src, dst):
    mesh = plsc.VectorSubcoreMesh(core_axis_name="c", subcore_axis_name="s")

    @functools.partial(
        pl.kernel,
        out_type=jax.ShapeDtypeStruct((N_HEAD, N, F_OUT), jnp.float32),
        mesh=mesh,
        compiler_params=pltpu.CompilerParams(needs_layout_passes=False),
        scratch_types=[
            pltpu.VMEM((N,), jnp.float32),
            pltpu.VMEM((N,), jnp.float32),
            pltpu.VMEM((1, B), jnp.int32),
            pltpu.VMEM((1, B), jnp.int32),
            pltpu.VMEM((1, B), jnp.int32),
            pltpu.VMEM((1, B), jnp.int32),
            pltpu.VMEM((B,), jnp.float32),
            pltpu.VMEM((B,), jnp.float32),
            pltpu.VMEM((B, C), jnp.float32),
            pltpu.VMEM((B, C), jnp.float32),
            pltpu.VMEM((16, C), jnp.float32),
            pltpu.VMEM((16, C), jnp.float32),
            pltpu.VMEM((LAST_SLICE,), jnp.float32),
            pltpu.VMEM((LAST_SLICE,), jnp.float32),
            pltpu.VMEM_SHARED((N, C), jnp.float32),
            pltpu.VMEM_SHARED((N,), jnp.float32),
            pltpu.SemaphoreType.DMA,
            pltpu.SemaphoreType.DMA,
            pltpu.SemaphoreType.DMA,
            pltpu.SemaphoreType.DMA,
        ],
    )
    def kern(hc_ref, al_ref, src_ref, dst_ref, out_ref,
             as_v, ad_v, sidx0, sidx1, didx0, didx1, wb0, wb1,
             rows0, rows1, zblk, fblk, zcol, rsv, acc, rsum,
             sem0, sem1, sems0, sems1):
        tid = lax.axis_index("s")
        core = lax.axis_index("c")
        ebase = tid * (E // 16)
        lo = tid * NODE_SLICE
        cnt = jnp.where(tid == 15, LAST_SLICE, NODE_SLICE)
        nz16 = cnt // 16

        zeros = jnp.zeros((16,), jnp.float32)
        for r in range(16):
            for q in range(C // L):
                zblk[r, pl.ds(q * L, L)] = zeros
        def zc_body(i, _):
            zcol[pl.ds(i * L, L)] = zeros
            return 0
        lax.fori_loop(0, LAST_SLICE // L, zc_body, 0)

        def load_idx(i, si, di):
            pltpu.sync_copy(src_ref.at[pl.ds(ebase + i * B, B)], si.at[0])
            pltpu.sync_copy(dst_ref.at[pl.ds(ebase + i * B, B)], di.at[0])

        def process(c, f, si, di, wb, rows, sems):
            @plsc.parallel_loop(0, B // L)
            def group(k):
                s16 = si[0, pl.ds(k * L, L)]
                d16 = di[0, pl.ds(k * L, L)]
                av = plsc.load_gather(as_v, [s16])
                dv = plsc.load_gather(ad_v, [d16])
                lg = av + dv
                lr = jnp.where(lg >= 0, lg, 0.2 * lg)
                w16 = jnp.exp(-lr)
                if f == 0:
                    wb[pl.ds(k * L, L)] = w16
                for j16 in range(L):
                    wj = _bcast16(w16, j16)
                    e = k * L + j16
                    for q in range(C // L):
                        rows[e, pl.ds(q * L, L)] = (
                            rows[e, pl.ds(q * L, L)] * wj)
            d = pltpu.async_copy(rows, acc.at[si.at[0]], sems, add=True)
            if f == 0:
                pltpu.sync_copy(wb, rsum.at[si.at[0]], add=True)
            return d

        for p in range(4):
            c = core * 4 + p
            h = c // 2
            f = p % 2

            if f == 0:
                pltpu.sync_copy(al_ref.at[2 * h], as_v)
                pltpu.sync_copy(al_ref.at[2 * h + 1], ad_v)

            def zero_body(i, _):
                pltpu.sync_copy(zblk, acc.at[pl.ds(lo + i * 16, 16)])
                return 0
            lax.fori_loop(0, nz16, zero_body, 0)
            if f == 0:
                pltpu.sync_copy(zcol.at[pl.ds(0, NODE_SLICE)],
                                rsum.at[pl.ds(lo, NODE_SLICE)])

                @pl.when(tid == 15)
                def _():
                    pltpu.sync_copy(
                        zcol.at[pl.ds(0, LAST_SLICE - NODE_SLICE)],
                        rsum.at[pl.ds(16 * NODE_SLICE,
                                      LAST_SLICE - NODE_SLICE)])
            plsc.subcore_barrier()

            load_idx(0, sidx0, didx0)

            def pair_body(i2, _):
                j = i2 * 2
                load_idx(j + 1, sidx1, didx1)
                g0 = pltpu.async_copy(hc_ref.at[c].at[didx0.at[0]],
                                      rows0, sem0)
                g1 = pltpu.async_copy(hc_ref.at[c].at[didx1.at[0]],
                                      rows1, sem1)
                g0.wait()
                d0 = process(c, f, sidx0, didx0, wb0, rows0, sems0)
                g1.wait()
                d1 = process(c, f, sidx1, didx1, wb1, rows1, sems1)
                d0.wait()
                load_idx(j + 2, sidx0, didx0)
                d1.wait()
                return 0
            lax.fori_loop(0, (NB - 1) // 2, pair_body, 0)
            ge = pltpu.async_copy(hc_ref.at[c].at[didx0.at[0]], rows0, sem0)
            ge.wait()
            process(c, f, sidx0, didx0, wb0, rows0, sems0).wait()
            plsc.subcore_barrier()

            pltpu.sync_copy(rsum.at[pl.ds(lo, NODE_SLICE)],
                            rsv.at[pl.ds(0, NODE_SLICE)])
            @pl.when(tid == 15)
            def _():
                pltpu.sync_copy(
                    rsum.at[pl.ds(16 * NODE_SLICE,
                                  LAST_SLICE - NODE_SLICE)],
                    rsv.at[pl.ds(NODE_SLICE, LAST_SLICE - NODE_SLICE)])

            def flush_body(i, _):
                pltpu.sync_copy(acc.at[pl.ds(lo + i * 16, 16)], fblk)
                r16 = 1.0 / rsv[pl.ds(i * L, L)]
                for j16 in range(L):
                    rj = _bcast16(r16, j16)
                    for q in range(C // L):
                        fblk[j16, pl.ds(q * L, L)] = (
                            fblk[j16, pl.ds(q * L, L)] * rj)
                pltpu.sync_copy(
                    fblk,
                    out_ref.at[h].at[pl.ds(lo + i * 16, 16),
                                     pl.ds(f * C, C)])
                return 0
            lax.fori_loop(0, nz16, flush_body, 0)
            plsc.subcore_barrier()

    return kern(hc, alphas, src, dst)


def kernel(input, edge_index, w, a_src_dst):
    x = input
    wc = jnp.transpose(w, (1, 0, 2)).reshape(F_IN, N_HEAD * F_OUT)
    a_src = a_src_dst[:, :F_OUT, 0]
    a_dst = a_src_dst[:, F_OUT:, 0]
    bs = jnp.einsum("hij,hj->hi", w, a_src)
    bd = jnp.einsum("hij,hj->hi", w, a_dst)
    bsd = jnp.stack([bs, bd], axis=1).reshape(2 * N_HEAD, F_IN)

    hc = pl.pallas_call(
        _matmul_chunks_kernel,
        grid=(NCHUNK, N // 1000),
        in_specs=[
            pl.BlockSpec((1000, F_IN), lambda c, n: (n, 0)),
            pl.BlockSpec((F_IN, C), lambda c, n: (0, c)),
        ],
        out_specs=pl.BlockSpec((1, 1000, C), lambda c, n: (c, n, 0)),
        out_shape=jax.ShapeDtypeStruct((NCHUNK, N, C), jnp.float32),
    )(x, wc)

    alphas = pl.pallas_call(
        _alphas_kernel,
        in_specs=[
            pl.BlockSpec((2 * N_HEAD, F_IN), lambda: (0, 0)),
            pl.BlockSpec((N, F_IN), lambda: (0, 0)),
        ],
        out_specs=pl.BlockSpec((2 * N_HEAD, N), lambda: (0, 0)),
        out_shape=jax.ShapeDtypeStruct((2 * N_HEAD, N), jnp.float32),
    )(bsd, x)

    return _sc_gat(hc, alphas, edge_index[0], edge_index[1])


# speedup vs baseline: 5.0409x; 1.2190x over previous
import functools

import jax
import jax.numpy as jnp
from jax import lax
from jax.experimental import pallas as pl
from jax.experimental.pallas import tpu as pltpu
from jax.experimental.pallas import tpu_sc as plsc

N = 10000
E = 160000
F_IN = 256
F_OUT = 256
N_HEAD = 4
C = 128
NCHUNK = (N_HEAD * F_OUT) // C
B = 80
L = 16
NB = (E // 16) // B

NODE_SLICE = 624
LAST_SLICE = N - 15 * NODE_SLICE
ZR = 24


def _matmul_chunks_kernel(x_ref, w_ref, o_ref):
    o_ref[0] = jnp.dot(x_ref[...], w_ref[...],
                       preferred_element_type=jnp.float32)


def _alphas_kernel(b_ref, x_ref, o_ref):
    o_ref[...] = lax.dot_general(b_ref[...], x_ref[...],
                                 (((1,), (1,)), ((), ())),
                                 preferred_element_type=jnp.float32)


def _bcast16(v, j):
    idx = jnp.full((16,), j, dtype=jnp.int32)
    dnums = lax.GatherDimensionNumbers(
        offset_dims=(), collapsed_slice_dims=(0,), start_index_map=(0,))
    return lax.gather(v, idx[:, None], dnums, (1,),
                      mode=lax.GatherScatterMode.PROMISE_IN_BOUNDS)


def _sc_gat(hc, alphas, src, dst):
    mesh = plsc.VectorSubcoreMesh(core_axis_name="c", subcore_axis_name="s")

    @functools.partial(
        pl.kernel,
        out_type=jax.ShapeDtypeStruct((N_HEAD, N, F_OUT), jnp.float32),
        mesh=mesh,
        compiler_params=pltpu.CompilerParams(needs_layout_passes=False),
        scratch_types=[
            pltpu.VMEM((N,), jnp.float32),
            pltpu.VMEM((N,), jnp.float32),
            pltpu.VMEM((1, B), jnp.int32),
            pltpu.VMEM((1, B), jnp.int32),
            pltpu.VMEM((1, B), jnp.int32),
            pltpu.VMEM((1, B), jnp.int32),
            pltpu.VMEM((B,), jnp.float32),
            pltpu.VMEM((B,), jnp.float32),
            pltpu.VMEM((B, C), jnp.float32),
            pltpu.VMEM((B, C), jnp.float32),
            pltpu.VMEM((16, C), jnp.float32),
            pltpu.VMEM((16, C), jnp.float32),
            pltpu.VMEM((LAST_SLICE,), jnp.float32),
            pltpu.VMEM((LAST_SLICE,), jnp.float32),
            pltpu.VMEM_SHARED((N, C), jnp.float32),
            pltpu.VMEM_SHARED((N,), jnp.float32),
            pltpu.SemaphoreType.DMA,
            pltpu.SemaphoreType.DMA,
            pltpu.SemaphoreType.DMA,
            pltpu.SemaphoreType.DMA,
            pltpu.SemaphoreType.DMA,
            pltpu.SemaphoreType.DMA,
        ],
    )
    def kern(hc_ref, al_ref, src_ref, dst_ref, out_ref,
             as_v, ad_v, sidx0, sidx1, didx0, didx1, wb0, wb1,
             rows0, rows1, zblk, fblk, zcol, rsv, acc, rsum,
             sem0, sem1, sems0, sems1, semr0, semr1):
        tid = lax.axis_index("s")
        core = lax.axis_index("c")
        ebase = tid * (E // 16)
        lo = tid * NODE_SLICE
        cnt = jnp.where(tid == 15, LAST_SLICE, NODE_SLICE)
        nz16 = cnt // 16

        zeros = jnp.zeros((16,), jnp.float32)
        for r in range(16):
            for q in range(C // L):
                zblk[r, pl.ds(q * L, L)] = zeros
        def zc_body(i, _):
            zcol[pl.ds(i * L, L)] = zeros
            return 0
        lax.fori_loop(0, LAST_SLICE // L, zc_body, 0)

        def load_idx(i, si, di):
            pltpu.sync_copy(src_ref.at[pl.ds(ebase + i * B, B)], si.at[0])
            pltpu.sync_copy(dst_ref.at[pl.ds(ebase + i * B, B)], di.at[0])

        def process(c, f, si, di, wb, rows, sems, semr):
            @plsc.parallel_loop(0, B // L)
            def group(k):
                s16 = si[0, pl.ds(k * L, L)]
                d16 = di[0, pl.ds(k * L, L)]
                av = plsc.load_gather(as_v, [s16])
                dv = plsc.load_gather(ad_v, [d16])
                lg = av + dv
                lr = jnp.where(lg >= 0, lg, 0.2 * lg)
                w16 = jnp.exp(-lr)
                if f == 0:
                    wb[pl.ds(k * L, L)] = w16
                for j16 in range(L):
                    wj = _bcast16(w16, j16)
                    e = k * L + j16
                    for q in range(C // L):
                        rows[e, pl.ds(q * L, L)] = (
                            rows[e, pl.ds(q * L, L)] * wj)
            d = pltpu.async_copy(rows, acc.at[si.at[0]], sems, add=True)
            dr = None
            if f == 0:
                dr = pltpu.async_copy(wb, rsum.at[si.at[0]], semr, add=True)
            return d, dr

        for p in range(4):
            c = core * 4 + p
            h = c // 2
            f = p % 2

            if f == 0:
                pltpu.sync_copy(al_ref.at[2 * h], as_v)
                pltpu.sync_copy(al_ref.at[2 * h + 1], ad_v)

            def zero_body(i, _):
                pltpu.sync_copy(zblk, acc.at[pl.ds(lo + i * 16, 16)])
                return 0
            lax.fori_loop(0, nz16, zero_body, 0)
            if f == 0:
                pltpu.sync_copy(zcol.at[pl.ds(0, NODE_SLICE)],
                                rsum.at[pl.ds(lo, NODE_SLICE)])

                @pl.when(tid == 15)
                def _():
                    pltpu.sync_copy(
                        zcol.at[pl.ds(0, LAST_SLICE - NODE_SLICE)],
                        rsum.at[pl.ds(16 * NODE_SLICE,
                                      LAST_SLICE - NODE_SLICE)])
            plsc.subcore_barrier()

            load_idx(0, sidx0, didx0)

            def pair_body(i2, _):
                j = i2 * 2
                g0 = pltpu.async_copy(hc_ref.at[c].at[didx0.at[0]],
                                      rows0, sem0)
                load_idx(j + 1, sidx1, didx1)
                g1 = pltpu.async_copy(hc_ref.at[c].at[didx1.at[0]],
                                      rows1, sem1)
                g0.wait()
                d0, dr0 = process(c, f, sidx0, didx0, wb0, rows0, sems0, semr0)
                g1.wait()
                d1, dr1 = process(c, f, sidx1, didx1, wb1, rows1, sems1, semr1)
                d0.wait()
                if f == 0:
                    dr0.wait()
                load_idx(j + 2, sidx0, didx0)
                d1.wait()
                if f == 0:
                    dr1.wait()
                return 0
            lax.fori_loop(0, (NB - 1) // 2, pair_body, 0)
            ge = pltpu.async_copy(hc_ref.at[c].at[didx0.at[0]], rows0, sem0)
            ge.wait()
            de, dre = process(c, f, sidx0, didx0, wb0, rows0, sems0, semr0)
            de.wait()
            if f == 0:
                dre.wait()
            plsc.subcore_barrier()

            pltpu.sync_copy(rsum.at[pl.ds(lo, NODE_SLICE)],
                            rsv.at[pl.ds(0, NODE_SLICE)])
            @pl.when(tid == 15)
            def _():
                pltpu.sync_copy(
                    rsum.at[pl.ds(16 * NODE_SLICE,
                                  LAST_SLICE - NODE_SLICE)],
                    rsv.at[pl.ds(NODE_SLICE, LAST_SLICE - NODE_SLICE)])

            def flush_body(i, _):
                pltpu.sync_copy(acc.at[pl.ds(lo + i * 16, 16)], fblk)
                r16 = 1.0 / rsv[pl.ds(i * L, L)]
                for j16 in range(L):
                    rj = _bcast16(r16, j16)
                    for q in range(C // L):
                        fblk[j16, pl.ds(q * L, L)] = (
                            fblk[j16, pl.ds(q * L, L)] * rj)
                pltpu.sync_copy(
                    fblk,
                    out_ref.at[h].at[pl.ds(lo + i * 16, 16),
                                     pl.ds(f * C, C)])
                return 0
            lax.fori_loop(0, nz16, flush_body, 0)
            plsc.subcore_barrier()

    return kern(hc, alphas, src, dst)


def kernel(input, edge_index, w, a_src_dst):
    x = input
    wc = jnp.transpose(w, (1, 0, 2)).reshape(F_IN, N_HEAD * F_OUT)
    a_src = a_src_dst[:, :F_OUT, 0]
    a_dst = a_src_dst[:, F_OUT:, 0]
    bs = jnp.einsum("hij,hj->hi", w, a_src)
    bd = jnp.einsum("hij,hj->hi", w, a_dst)
    bsd = jnp.stack([bs, bd], axis=1).reshape(2 * N_HEAD, F_IN)

    hc = pl.pallas_call(
        _matmul_chunks_kernel,
        grid=(NCHUNK, N // 1000),
        in_specs=[
            pl.BlockSpec((1000, F_IN), lambda c, n: (n, 0)),
            pl.BlockSpec((F_IN, C), lambda c, n: (0, c)),
        ],
        out_specs=pl.BlockSpec((1, 1000, C), lambda c, n: (c, n, 0)),
        out_shape=jax.ShapeDtypeStruct((NCHUNK, N, C), jnp.float32),
    )(x, wc)

    alphas = pl.pallas_call(
        _alphas_kernel,
        in_specs=[
            pl.BlockSpec((2 * N_HEAD, F_IN), lambda: (0, 0)),
            pl.BlockSpec((N, F_IN), lambda: (0, 0)),
        ],
        out_specs=pl.BlockSpec((2 * N_HEAD, N), lambda: (0, 0)),
        out_shape=jax.ShapeDtypeStruct((2 * N_HEAD, N), jnp.float32),
    )(bsd, x)

    return _sc_gat(hc, alphas, edge_index[0], edge_index[1])


# speedup vs baseline: 5.4985x; 1.0908x over previous
import functools

import jax
import jax.numpy as jnp
from jax import lax
from jax.experimental import pallas as pl
from jax.experimental.pallas import tpu as pltpu
from jax.experimental.pallas import tpu_sc as plsc

N = 10000
E = 160000
F_IN = 256
F_OUT = 256
N_HEAD = 4
C = 128
NCHUNK = (N_HEAD * F_OUT) // C
B = 80
L = 16
NB = (E // 16) // B

NODE_SLICE = 624
LAST_SLICE = N - 15 * NODE_SLICE
ZR = 24


def _matmul_chunks_kernel(x_ref, w_ref, o_ref):
    o_ref[0] = jnp.dot(x_ref[...], w_ref[...],
                       preferred_element_type=jnp.float32)


def _alphas_kernel(b_ref, x_ref, o_ref):
    o_ref[...] = lax.dot_general(b_ref[...], x_ref[...],
                                 (((1,), (1,)), ((), ())),
                                 preferred_element_type=jnp.float32)


def _bcast16(v, j):
    idx = jnp.full((16,), j, dtype=jnp.int32)
    dnums = lax.GatherDimensionNumbers(
        offset_dims=(), collapsed_slice_dims=(0,), start_index_map=(0,))
    return lax.gather(v, idx[:, None], dnums, (1,),
                      mode=lax.GatherScatterMode.PROMISE_IN_BOUNDS)


def _sc_gat(hc, alphas, src2d, dst2d, z2d):
    mesh = plsc.VectorSubcoreMesh(core_axis_name="c", subcore_axis_name="s")

    @functools.partial(
        pl.kernel,
        out_type=jax.ShapeDtypeStruct((N_HEAD, N, F_OUT), jnp.float32),
        mesh=mesh,
        compiler_params=pltpu.CompilerParams(needs_layout_passes=False),
        scratch_types=[
            pltpu.VMEM((N,), jnp.float32),
            pltpu.VMEM((N,), jnp.float32),
            pltpu.VMEM((25, B), jnp.int32),
            pltpu.VMEM((25, B), jnp.int32),
            pltpu.VMEM((B,), jnp.float32),
            pltpu.VMEM((B,), jnp.float32),
            pltpu.VMEM((B, C), jnp.float32),
            pltpu.VMEM((B, C), jnp.float32),
            pltpu.VMEM((LAST_SLICE,), jnp.float32),
            pltpu.VMEM_SHARED((N, C), jnp.float32),
            pltpu.VMEM_SHARED((N,), jnp.float32),
            pltpu.SemaphoreType.DMA,
            pltpu.SemaphoreType.DMA,
            pltpu.SemaphoreType.DMA,
            pltpu.SemaphoreType.DMA,
            pltpu.SemaphoreType.DMA,
            pltpu.SemaphoreType.DMA,
        ],
    )
    def kern(hc_ref, al_ref, src_ref, dst_ref, z2_ref, out_ref,
             as_v, ad_v, sidx_ch, didx_ch, wb0, wb1,
             rows0, rows1, rsv, acc, rsum,
             sem0, sem1, sems0, sems1, semr0, semr1):
        tid = lax.axis_index("s")
        core = lax.axis_index("c")
        lo = tid * NODE_SLICE
        cnt = jnp.where(tid == 15, LAST_SLICE, NODE_SLICE)
        nz16 = cnt // 16

        def process(c, f, r, wb, rows, sems, semr):
            @plsc.parallel_loop(0, B // L)
            def group(k):
                s16 = sidx_ch[r, pl.ds(k * L, L)]
                d16 = didx_ch[r, pl.ds(k * L, L)]
                av = plsc.load_gather(as_v, [s16])
                dv = plsc.load_gather(ad_v, [d16])
                lg = av + dv
                lr = jnp.where(lg >= 0, lg, 0.2 * lg)
                w16 = jnp.exp(-lr)
                if f == 0:
                    wb[pl.ds(k * L, L)] = w16
                for j16 in range(L):
                    wj = _bcast16(w16, j16)
                    e = k * L + j16
                    for q in range(C // L):
                        rows[e, pl.ds(q * L, L)] = (
                            rows[e, pl.ds(q * L, L)] * wj)
            d = pltpu.async_copy(rows, acc.at[sidx_ch.at[r]], sems, add=True)
            dr = None
            if f == 0:
                dr = pltpu.async_copy(wb, rsum.at[sidx_ch.at[r]], semr,
                                      add=True)
            return d, dr

        for p in range(4):
            c = core * 4 + p
            h = c // 2
            f = p % 2

            if f == 0:
                pltpu.sync_copy(al_ref.at[2 * h], as_v)
                pltpu.sync_copy(al_ref.at[2 * h + 1], ad_v)

            pltpu.sync_copy(z2_ref.at[pl.ds(0, NODE_SLICE)],
                            acc.at[pl.ds(lo, NODE_SLICE)])
            @pl.when(tid == 15)
            def _():
                pltpu.sync_copy(
                    z2_ref.at[pl.ds(0, LAST_SLICE - NODE_SLICE)],
                    acc.at[pl.ds(16 * NODE_SLICE,
                                 LAST_SLICE - NODE_SLICE)])
            if f == 0:
                zeros = jnp.zeros((16,), jnp.float32)
                def zr_body(i, _):
                    rsv[pl.ds(i * L, L)] = zeros
                    return 0
                lax.fori_loop(0, LAST_SLICE // L, zr_body, 0)
                pltpu.sync_copy(rsv.at[pl.ds(0, NODE_SLICE)],
                                rsum.at[pl.ds(lo, NODE_SLICE)])

                @pl.when(tid == 15)
                def _():
                    pltpu.sync_copy(
                        rsv.at[pl.ds(0, LAST_SLICE - NODE_SLICE)],
                        rsum.at[pl.ds(16 * NODE_SLICE,
                                      LAST_SLICE - NODE_SLICE)])
            plsc.subcore_barrier()

            def chunk_body(ch, _):
                pltpu.sync_copy(src_ref.at[tid].at[ch], sidx_ch)
                pltpu.sync_copy(dst_ref.at[tid].at[ch], didx_ch)

                def pair_body(i2, _):
                    r0 = i2 * 2
                    r1 = r0 + 1
                    g0 = pltpu.async_copy(hc_ref.at[c].at[didx_ch.at[r0]],
                                          rows0, sem0)
                    g1 = pltpu.async_copy(hc_ref.at[c].at[didx_ch.at[r1]],
                                          rows1, sem1)
                    g0.wait()
                    d0, dr0 = process(c, f, r0, wb0, rows0, sems0, semr0)
                    g1.wait()
                    d1, dr1 = process(c, f, r1, wb1, rows1, sems1, semr1)
                    d0.wait()
                    d1.wait()
                    if f == 0:
                        dr0.wait()
                        dr1.wait()
                    return 0
                lax.fori_loop(0, 12, pair_body, 0)
                ge = pltpu.async_copy(hc_ref.at[c].at[didx_ch.at[24]],
                                      rows0, sem0)
                ge.wait()
                de, dre = process(c, f, 24, wb0, rows0, sems0, semr0)
                de.wait()
                if f == 0:
                    dre.wait()
                return 0
            lax.fori_loop(0, 5, chunk_body, 0)
            plsc.subcore_barrier()

            pltpu.sync_copy(rsum.at[pl.ds(lo, NODE_SLICE)],
                            rsv.at[pl.ds(0, NODE_SLICE)])
            @pl.when(tid == 15)
            def _():
                pltpu.sync_copy(
                    rsum.at[pl.ds(16 * NODE_SLICE,
                                  LAST_SLICE - NODE_SLICE)],
                    rsv.at[pl.ds(NODE_SLICE, LAST_SLICE - NODE_SLICE)])

            def flush_body(i, _):
                pltpu.sync_copy(acc.at[pl.ds(lo + i * 16, 16)],
                                rows0.at[pl.ds(0, 16)])
                r16 = 1.0 / rsv[pl.ds(i * L, L)]
                for j16 in range(L):
                    rj = _bcast16(r16, j16)
                    for q in range(C // L):
                        rows0[j16, pl.ds(q * L, L)] = (
                            rows0[j16, pl.ds(q * L, L)] * rj)
                pltpu.sync_copy(
                    rows0.at[pl.ds(0, 16)],
                    out_ref.at[h].at[pl.ds(lo + i * 16, 16),
                                     pl.ds(f * C, C)])
                return 0
            lax.fori_loop(0, nz16, flush_body, 0)
            plsc.subcore_barrier()

    return kern(hc, alphas, src2d, dst2d, z2d)


def kernel(input, edge_index, w, a_src_dst):
    x = input
    wc = jnp.transpose(w, (1, 0, 2)).reshape(F_IN, N_HEAD * F_OUT)
    a_src = a_src_dst[:, :F_OUT, 0]
    a_dst = a_src_dst[:, F_OUT:, 0]
    bs = jnp.einsum("hij,hj->hi", w, a_src)
    bd = jnp.einsum("hij,hj->hi", w, a_dst)
    bsd = jnp.stack([bs, bd], axis=1).reshape(2 * N_HEAD, F_IN)

    hc = pl.pallas_call(
        _matmul_chunks_kernel,
        grid=(NCHUNK, N // 1000),
        in_specs=[
            pl.BlockSpec((1000, F_IN), lambda c, n: (n, 0)),
            pl.BlockSpec((F_IN, C), lambda c, n: (0, c)),
        ],
        out_specs=pl.BlockSpec((1, 1000, C), lambda c, n: (c, n, 0)),
        out_shape=jax.ShapeDtypeStruct((NCHUNK, N, C), jnp.float32),
    )(x, wc)

    alphas = pl.pallas_call(
        _alphas_kernel,
        in_specs=[
            pl.BlockSpec((2 * N_HEAD, F_IN), lambda: (0, 0)),
            pl.BlockSpec((N, F_IN), lambda: (0, 0)),
        ],
        out_specs=pl.BlockSpec((2 * N_HEAD, N), lambda: (0, 0)),
        out_shape=jax.ShapeDtypeStruct((2 * N_HEAD, N), jnp.float32),
    )(bsd, x)

    src4d = edge_index[0].reshape(16, 5, 25, B)
    dst4d = edge_index[1].reshape(16, 5, 25, B)
    z2d = jnp.zeros((LAST_SLICE, C), jnp.float32)
    return _sc_gat(hc, alphas, src4d, dst4d, z2d)
